# baseline (speedup 1.0000x reference)
import functools

import jax
import jax.numpy as jnp
from jax import lax
from jax.experimental import pallas as pl
from jax.experimental.pallas import tpu as pltpu
from jax.experimental.pallas import tpu_sc as plsc

NN = 10000
DD = 128
N8 = 10112
ROWS_PER_TILE = N8 // 16
CHUNK = 128
NCHUNK = 80
E_PAD = 32 * NCHUNK * CHUNK
FF = 128
DEG_W = 128

_PREC = jax.lax.Precision.HIGHEST


@functools.cache
def _deg_kernel(deg_w=DEG_W):
    mesh = plsc.VectorSubcoreMesh(core_axis_name="c", subcore_axis_name="s")

    @functools.partial(
        pl.kernel,
        out_type=jax.ShapeDtypeStruct((2, N8, deg_w), jnp.float32),
        mesh=mesh,
        scratch_types=[
            pltpu.VMEM((NCHUNK, CHUNK), jnp.int32),
            pltpu.VMEM((CHUNK, deg_w), jnp.float32),
            pltpu.VMEM_SHARED((N8, deg_w), jnp.float32),
        ],
    )
    def deg_kernel(dst_hbm, ones_hbm, z_hbm, out_hbm, dst_v, ones_v, acc_sh):
        c = lax.axis_index("c")
        s = lax.axis_index("s")
        row0 = s * ROWS_PER_TILE
        pltpu.sync_copy(z_hbm.at[pl.ds(row0, ROWS_PER_TILE)],
                        acc_sh.at[pl.ds(row0, ROWS_PER_TILE)])
        pltpu.sync_copy(ones_hbm, ones_v)
        w = c * 16 + s
        pltpu.sync_copy(dst_hbm.at[pl.ds(w * NCHUNK, NCHUNK)], dst_v)
        plsc.subcore_barrier()

        @pl.loop(0, NCHUNK)
        def _(j):
            pltpu.sync_copy(ones_v, acc_sh.at[dst_v.at[j]], add=True)

        plsc.subcore_barrier()
        pltpu.sync_copy(acc_sh.at[pl.ds(row0, ROWS_PER_TILE)],
                        out_hbm.at[c, pl.ds(row0, ROWS_PER_TILE)])

    return deg_kernel


CH_C0 = 80
CH_C1 = 80
STAGE = 40
IDX_ROWS = 16 * (CH_C0 + CH_C1)


@functools.cache
def _agg_kernel():
    mesh = plsc.VectorSubcoreMesh(core_axis_name="c", subcore_axis_name="s")

    @functools.partial(
        pl.kernel,
        out_type=jax.ShapeDtypeStruct((2, N8, FF), jnp.float32),
        mesh=mesh,
        scratch_types=[
            pltpu.VMEM((STAGE, CHUNK), jnp.int32),
            pltpu.VMEM((STAGE, CHUNK), jnp.int32),
            pltpu.VMEM((2, CHUNK, FF), jnp.float32),
            pltpu.VMEM_SHARED((N8, FF), jnp.float32),
            pltpu.SemaphoreType.DMA,
            pltpu.SemaphoreType.DMA,
        ],
    )
    def agg_kernel(g_hbm, src_hbm, dst_hbm, z_hbm, out_hbm,
                   src_v, dst_v, rows_v, acc_sh, sem0, sem1):
        c = lax.axis_index("c")
        s = lax.axis_index("s")
        row0 = s * ROWS_PER_TILE
        with jax.named_scope("acc_zero"):
            pltpu.sync_copy(z_hbm.at[pl.ds(row0, ROWS_PER_TILE)],
                            acc_sh.at[pl.ds(row0, ROWS_PER_TILE)])
        base = lax.select(c == 0, s * CH_C0, 16 * CH_C0 + s * CH_C1)
        with jax.named_scope("zero_barrier"):
            plsc.subcore_barrier()

        sems = (sem0, sem1)

        def gather(j, b):
            return pltpu.async_copy(
                g_hbm.at[src_v.at[j]], rows_v.at[b], sems[b])

        def wait(j, b):
            pltpu.make_async_copy(
                g_hbm.at[src_v.at[j]], rows_v.at[b], sems[b]).wait()

        def scat(b, j):
            pltpu.sync_copy(rows_v.at[b], acc_sh.at[dst_v.at[j]],
                            add=True)

        def run(npasses):
            for p in range(npasses):
                sb = base + p * STAGE
                with jax.named_scope("stage_idx"):
                    pltpu.sync_copy(src_hbm.at[pl.ds(sb, STAGE)], src_v)
                    pltpu.sync_copy(dst_hbm.at[pl.ds(sb, STAGE)], dst_v)
                gather(0, 0)

                @pl.loop(0, STAGE // 2)
                def _(i):
                    j0 = 2 * i
                    wait(j0, 0)
                    gather(j0 + 1, 1)
                    scat(0, j0)
                    wait(j0 + 1, 1)

                    @pl.when(i < STAGE // 2 - 1)
                    def _():
                        gather(j0 + 2, 0)

                    scat(1, j0 + 1)

        with jax.named_scope("edge_loop"):
            @pl.when(c == 0)
            def _():
                run(CH_C0 // STAGE)

            @pl.when(c == 1)
            def _():
                run(CH_C1 // STAGE)

        with jax.named_scope("end_barrier"):
            plsc.subcore_barrier()
        with jax.named_scope("copyout"):
            pltpu.sync_copy(acc_sh.at[pl.ds(row0, ROWS_PER_TILE)],
                            out_hbm.at[c, pl.ds(row0, ROWS_PER_TILE)])

    return agg_kernel


def _tc_first_body(x_ref, w_ref, degp_ref, g_ref, dinv_ref):
    deg = 1.0 + degp_ref[0][:, 0:1] + degp_ref[1][:, 0:1]
    dinv = jax.lax.rsqrt(deg)
    p = jnp.dot(x_ref[...], w_ref[...],
                preferred_element_type=jnp.float32, precision=_PREC)
    g_ref[...] = dinv * p
    dinv_ref[...] = jnp.broadcast_to(dinv, (N8, 8))


def _tc_first(xp, w1p, degp):
    return pl.pallas_call(
        _tc_first_body,
        out_shape=(jax.ShapeDtypeStruct((N8, FF), jnp.float32),
                   jax.ShapeDtypeStruct((N8, 8), jnp.float32)),
    )(xp, w1p, degp)


def _tc_mid_body(pp_ref, g_ref, dinv_ref, b_ref, w_ref, out_ref):
    dinv = dinv_ref[...][:, 0:1]
    agg = pp_ref[0] + pp_ref[1] + g_ref[...]
    h = jnp.maximum(dinv * agg + b_ref[0:1, :], 0.0)
    out_ref[...] = dinv * jnp.dot(h, w_ref[...],
                                  preferred_element_type=jnp.float32,
                                  precision=_PREC)


def _tc_mid(pp, g, dinv, bp, wp):
    return pl.pallas_call(
        _tc_mid_body,
        out_shape=jax.ShapeDtypeStruct((N8, FF), jnp.float32),
    )(pp, g, dinv, bp, wp)


def _tc_final_body(pp_ref, g_ref, dinv_ref, b_ref, w_ref, bl_ref, out_ref):
    dinv = dinv_ref[...][:, 0:1]
    agg = pp_ref[0] + pp_ref[1] + g_ref[...]
    h = jnp.maximum(dinv * agg + b_ref[0:1, :], 0.0)
    out_ref[...] = jnp.dot(h, w_ref[...],
                           preferred_element_type=jnp.float32,
                           precision=_PREC) + bl_ref[0:1, :]


def _tc_final(pp, g, dinv, bp, wlp, blp):
    return pl.pallas_call(
        _tc_final_body,
        out_shape=jax.ShapeDtypeStruct((N8, 8), jnp.float32),
    )(pp, g, dinv, bp, wlp, blp)


def _padw(W, fi, fo):
    return jnp.zeros((fi, fo), jnp.float32).at[:W.shape[0], :W.shape[1]].set(W)


def _padb(b, fo):
    return jnp.broadcast_to(jnp.pad(b, (0, fo - b.shape[0])), (8, fo))


def kernel(x, edge_index, W1, b1, W2, b2, W6, b6, W3, b3, W4, b4, Wl, bl):
    f32 = jnp.float32
    src = edge_index[0]
    dst = edge_index[1]
    e = src.shape[0]
    pad = IDX_ROWS * CHUNK - e
    pad_dst = NN + (jnp.arange(pad, dtype=jnp.int32) % (N8 - NN))
    src2d = jnp.concatenate(
        [src, jnp.zeros((pad,), jnp.int32)]).reshape(IDX_ROWS, CHUNK)
    dst2d = jnp.concatenate(
        [dst, pad_dst]).reshape(IDX_ROWS, CHUNK)
    xp = jnp.zeros((N8, DD), f32).at[:NN].set(x)

    w1p = _padw(W1, DD, FF)
    w2p = _padw(W2, FF, FF)
    w6p = _padw(W6, FF, FF)
    w3p = _padw(W3, FF, FF)
    w4p = _padw(W4, FF, FF)
    wlp = _padw(Wl, FF, 8)
    b1p = _padb(b1, FF)
    b2p = _padb(b2, FF)
    b6p = _padb(b6, FF)
    b3p = _padb(b3, FF)
    b4p = _padb(b4, FF)
    blp = _padb(bl, 8)

    ones = jnp.ones((CHUNK, DEG_W), f32)
    zdeg = jnp.zeros((N8, DEG_W), f32)
    zf = jnp.zeros((N8, FF), f32)

    degp = _deg_kernel()(dst2d, ones, zdeg)
    g1, dinv = _tc_first(xp, w1p, degp)

    agg = _agg_kernel()
    p1 = agg(g1, src2d, dst2d, zf)
    g2 = _tc_mid(p1, g1, dinv, b1p, w2p)
    p2 = agg(g2, src2d, dst2d, zf)
    g3 = _tc_mid(p2, g2, dinv, b2p, w6p)
    p3 = agg(g3, src2d, dst2d, zf)
    g4 = _tc_mid(p3, g3, dinv, b6p, w3p)
    p4 = agg(g4, src2d, dst2d, zf)
    g5 = _tc_mid(p4, g4, dinv, b3p, w4p)
    p5 = agg(g5, src2d, dst2d, zf)
    out = _tc_final(p5, g5, dinv, b4p, wlp, blp)
    return out[:NN, :4]


# speedup vs baseline: 3.1592x; 3.1592x over previous
import functools

import jax
import jax.numpy as jnp
from jax import lax
from jax.experimental import pallas as pl
from jax.experimental.pallas import tpu as pltpu
from jax.experimental.pallas import tpu_sc as plsc

NN = 10000
DD = 128
N8 = 10112
ROWS_PER_TILE = N8 // 16
CHUNK = 128
NCHUNK = 80
E_PAD = 32 * NCHUNK * CHUNK
FF = 128
DEG_W = 128

_PREC = jax.lax.Precision.HIGHEST


@functools.cache
def _deg_kernel(deg_w=DEG_W):
    mesh = plsc.VectorSubcoreMesh(core_axis_name="c", subcore_axis_name="s")

    @functools.partial(
        pl.kernel,
        out_type=jax.ShapeDtypeStruct((2, N8, deg_w), jnp.float32),
        mesh=mesh,
        scratch_types=[
            pltpu.VMEM((NCHUNK, CHUNK), jnp.int32),
            pltpu.VMEM((CHUNK, deg_w), jnp.float32),
            pltpu.VMEM_SHARED((N8, deg_w), jnp.float32),
        ],
    )
    def deg_kernel(dst_hbm, ones_hbm, z_hbm, out_hbm, dst_v, ones_v, acc_sh):
        c = lax.axis_index("c")
        s = lax.axis_index("s")
        row0 = s * ROWS_PER_TILE
        pltpu.sync_copy(z_hbm.at[pl.ds(row0, ROWS_PER_TILE)],
                        acc_sh.at[pl.ds(row0, ROWS_PER_TILE)])
        pltpu.sync_copy(ones_hbm, ones_v)
        w = c * 16 + s
        pltpu.sync_copy(dst_hbm.at[pl.ds(w * NCHUNK, NCHUNK)], dst_v)
        plsc.subcore_barrier()

        @pl.loop(0, NCHUNK)
        def _(j):
            pltpu.sync_copy(ones_v, acc_sh.at[dst_v.at[j]], add=True)

        plsc.subcore_barrier()
        pltpu.sync_copy(acc_sh.at[pl.ds(row0, ROWS_PER_TILE)],
                        out_hbm.at[c, pl.ds(row0, ROWS_PER_TILE)])

    return deg_kernel


CH_C0 = 80
CH_C1 = 80
STAGE = 40
IDX_ROWS = 16 * (CH_C0 + CH_C1)


@functools.cache
def _agg_kernel():
    mesh = plsc.VectorSubcoreMesh(core_axis_name="c", subcore_axis_name="s")

    @functools.partial(
        pl.kernel,
        out_type=jax.ShapeDtypeStruct((2, N8, FF), jnp.float32),
        mesh=mesh,
        scratch_types=[
            pltpu.VMEM((STAGE, CHUNK), jnp.int32),
            pltpu.VMEM((STAGE, CHUNK), jnp.int32),
            pltpu.VMEM((2, CHUNK, FF), jnp.float32),
            pltpu.VMEM_SHARED((N8, FF), jnp.float32),
            pltpu.SemaphoreType.DMA,
            pltpu.SemaphoreType.DMA,
        ],
    )
    def agg_kernel(g_hbm, src_hbm, dst_hbm, z_hbm, out_hbm,
                   src_v, dst_v, rows_v, acc_sh, sem0, sem1):
        c = lax.axis_index("c")
        s = lax.axis_index("s")
        row0 = s * ROWS_PER_TILE
        with jax.named_scope("acc_zero"):
            pltpu.sync_copy(z_hbm.at[pl.ds(row0, ROWS_PER_TILE)],
                            acc_sh.at[pl.ds(row0, ROWS_PER_TILE)])
        base = lax.select(c == 0, s * CH_C0, 16 * CH_C0 + s * CH_C1)
        with jax.named_scope("zero_barrier"):
            plsc.subcore_barrier()

        sems = (sem0, sem1)

        def gather(j, b):
            return pltpu.async_copy(
                g_hbm.at[src_v.at[j]], rows_v.at[b], sems[b])

        def wait(j, b):
            pltpu.make_async_copy(
                g_hbm.at[src_v.at[j]], rows_v.at[b], sems[b]).wait()

        def scat(b, j):
            pltpu.sync_copy(rows_v.at[b], acc_sh.at[dst_v.at[j]],
                            add=True)

        def run(npasses):
            for p in range(npasses):
                sb = base + p * STAGE
                with jax.named_scope("stage_idx"):
                    pltpu.sync_copy(src_hbm.at[pl.ds(sb, STAGE)], src_v)
                    pltpu.sync_copy(dst_hbm.at[pl.ds(sb, STAGE)], dst_v)
                gather(0, 0)

                @pl.loop(0, STAGE // 2)
                def _(i):
                    j0 = 2 * i
                    wait(j0, 0)
                    gather(j0 + 1, 1)
                    scat(0, j0)
                    wait(j0 + 1, 1)

                    @pl.when(i < STAGE // 2 - 1)
                    def _():
                        gather(j0 + 2, 0)

                    scat(1, j0 + 1)

        with jax.named_scope("edge_loop"):
            @pl.when(c == 0)
            def _():
                run(CH_C0 // STAGE)

            @pl.when(c == 1)
            def _():
                run(CH_C1 // STAGE)

        with jax.named_scope("end_barrier"):
            plsc.subcore_barrier()
        with jax.named_scope("copyout"):
            pltpu.sync_copy(acc_sh.at[pl.ds(row0, ROWS_PER_TILE)],
                            out_hbm.at[c, pl.ds(row0, ROWS_PER_TILE)])

    return agg_kernel


def _tc_first_body(x_ref, w_ref, degp_ref, g_ref, dinv_ref):
    deg = 1.0 + degp_ref[0][:, 0:1] + degp_ref[1][:, 0:1]
    dinv = jax.lax.rsqrt(deg)
    p = jnp.dot(x_ref[...], w_ref[...],
                preferred_element_type=jnp.float32, precision=_PREC)
    g_ref[...] = dinv * p
    dinv_ref[...] = jnp.broadcast_to(dinv, (N8, 8))


def _tc_first(xp, w1p, degp):
    return pl.pallas_call(
        _tc_first_body,
        out_shape=(jax.ShapeDtypeStruct((N8, FF), jnp.float32),
                   jax.ShapeDtypeStruct((N8, 8), jnp.float32)),
    )(xp, w1p, degp)


def _tc_mid_body(pp_ref, g_ref, dinv_ref, b_ref, w_ref, out_ref):
    dinv = dinv_ref[...][:, 0:1]
    agg = pp_ref[0] + pp_ref[1] + g_ref[...]
    h = jnp.maximum(dinv * agg + b_ref[0:1, :], 0.0)
    out_ref[...] = dinv * jnp.dot(h, w_ref[...],
                                  preferred_element_type=jnp.float32,
                                  precision=_PREC)


def _tc_mid(pp, g, dinv, bp, wp):
    return pl.pallas_call(
        _tc_mid_body,
        out_shape=jax.ShapeDtypeStruct((N8, FF), jnp.float32),
    )(pp, g, dinv, bp, wp)


def _tc_final_body(pp_ref, g_ref, dinv_ref, b_ref, w_ref, bl_ref, out_ref):
    dinv = dinv_ref[...][:, 0:1]
    agg = pp_ref[0] + pp_ref[1] + g_ref[...]
    h = jnp.maximum(dinv * agg + b_ref[0:1, :], 0.0)
    out_ref[...] = jnp.dot(h, w_ref[...],
                           preferred_element_type=jnp.float32,
                           precision=_PREC) + bl_ref[0:1, :]


def _tc_final(pp, g, dinv, bp, wlp, blp):
    return pl.pallas_call(
        _tc_final_body,
        out_shape=jax.ShapeDtypeStruct((N8, 8), jnp.float32),
    )(pp, g, dinv, bp, wlp, blp)


def _padw(W, fi, fo):
    return jnp.zeros((fi, fo), jnp.float32).at[:W.shape[0], :W.shape[1]].set(W)


def _padb(b, fo):
    return jnp.broadcast_to(jnp.pad(b, (0, fo - b.shape[0])), (8, fo))


def kernel(x, edge_index, W1, b1, W2, b2, W6, b6, W3, b3, W4, b4, Wl, bl):
    f32 = jnp.float32
    src = edge_index[0]
    dst = edge_index[1]
    e = src.shape[0]
    pad = IDX_ROWS * CHUNK - e
    pad_dst = NN + (jnp.arange(pad, dtype=jnp.int32) % (N8 - NN))
    pad_src = jnp.arange(pad, dtype=jnp.int32) % NN
    src2d = jnp.concatenate(
        [src, pad_src]).reshape(IDX_ROWS, CHUNK)
    dst2d = jnp.concatenate(
        [dst, pad_dst]).reshape(IDX_ROWS, CHUNK)
    xp = jnp.zeros((N8, DD), f32).at[:NN].set(x)

    w1p = _padw(W1, DD, FF)
    w2p = _padw(W2, FF, FF)
    w6p = _padw(W6, FF, FF)
    w3p = _padw(W3, FF, FF)
    w4p = _padw(W4, FF, FF)
    wlp = _padw(Wl, FF, 8)
    b1p = _padb(b1, FF)
    b2p = _padb(b2, FF)
    b6p = _padb(b6, FF)
    b3p = _padb(b3, FF)
    b4p = _padb(b4, FF)
    blp = _padb(bl, 8)

    ones = jnp.ones((CHUNK, DEG_W), f32)
    zdeg = jnp.zeros((N8, DEG_W), f32)
    zf = jnp.zeros((N8, FF), f32)

    degp = _deg_kernel()(dst2d, ones, zdeg)
    g1, dinv = _tc_first(xp, w1p, degp)

    agg = _agg_kernel()
    p1 = agg(g1, src2d, dst2d, zf)
    g2 = _tc_mid(p1, g1, dinv, b1p, w2p)
    p2 = agg(g2, src2d, dst2d, zf)
    g3 = _tc_mid(p2, g2, dinv, b2p, w6p)
    p3 = agg(g3, src2d, dst2d, zf)
    g4 = _tc_mid(p3, g3, dinv, b6p, w3p)
    p4 = agg(g4, src2d, dst2d, zf)
    g5 = _tc_mid(p4, g4, dinv, b3p, w4p)
    p5 = agg(g5, src2d, dst2d, zf)
    out = _tc_final(p5, g5, dinv, b4p, wlp, blp)
    return out[:NN, :4]


# speedup vs baseline: 3.2437x; 1.0268x over previous
import functools

import jax
import jax.numpy as jnp
from jax import lax
from jax.experimental import pallas as pl
from jax.experimental.pallas import tpu as pltpu
from jax.experimental.pallas import tpu_sc as plsc

NN = 10000
DD = 128
N8 = 10112
ROWS_PER_TILE = N8 // 16
CHUNK = 128
NCHUNK = 80
E_PAD = 32 * NCHUNK * CHUNK
FF = 128
DEG_W = 128

_PREC = None


@functools.cache
def _deg_kernel(deg_w=DEG_W):
    mesh = plsc.VectorSubcoreMesh(core_axis_name="c", subcore_axis_name="s")

    @functools.partial(
        pl.kernel,
        out_type=jax.ShapeDtypeStruct((2, N8, deg_w), jnp.float32),
        mesh=mesh,
        scratch_types=[
            pltpu.VMEM((NCHUNK, CHUNK), jnp.int32),
            pltpu.VMEM((CHUNK, deg_w), jnp.float32),
            pltpu.VMEM_SHARED((N8, deg_w), jnp.float32),
        ],
    )
    def deg_kernel(dst_hbm, ones_hbm, z_hbm, out_hbm, dst_v, ones_v, acc_sh):
        c = lax.axis_index("c")
        s = lax.axis_index("s")
        row0 = s * ROWS_PER_TILE
        pltpu.sync_copy(z_hbm.at[pl.ds(row0, ROWS_PER_TILE)],
                        acc_sh.at[pl.ds(row0, ROWS_PER_TILE)])
        pltpu.sync_copy(ones_hbm, ones_v)
        w = c * 16 + s
        pltpu.sync_copy(dst_hbm.at[pl.ds(w * NCHUNK, NCHUNK)], dst_v)
        plsc.subcore_barrier()

        @pl.loop(0, NCHUNK)
        def _(j):
            pltpu.sync_copy(ones_v, acc_sh.at[dst_v.at[j]], add=True)

        plsc.subcore_barrier()
        pltpu.sync_copy(acc_sh.at[pl.ds(row0, ROWS_PER_TILE)],
                        out_hbm.at[c, pl.ds(row0, ROWS_PER_TILE)])

    return deg_kernel


CH_C0 = 80
CH_C1 = 80
STAGE = 40
IDX_ROWS = 16 * (CH_C0 + CH_C1)


@functools.cache
def _agg_kernel():
    mesh = plsc.VectorSubcoreMesh(core_axis_name="c", subcore_axis_name="s")

    @functools.partial(
        pl.kernel,
        out_type=jax.ShapeDtypeStruct((2, N8, FF), jnp.float32),
        mesh=mesh,
        scratch_types=[
            pltpu.VMEM((STAGE, CHUNK), jnp.int32),
            pltpu.VMEM((STAGE, CHUNK), jnp.int32),
            pltpu.VMEM((2, CHUNK, FF), jnp.float32),
            pltpu.VMEM_SHARED((N8, FF), jnp.float32),
            pltpu.SemaphoreType.DMA,
            pltpu.SemaphoreType.DMA,
        ],
    )
    def agg_kernel(g_hbm, src_hbm, dst_hbm, z_hbm, out_hbm,
                   src_v, dst_v, rows_v, acc_sh, sem0, sem1):
        c = lax.axis_index("c")
        s = lax.axis_index("s")
        row0 = s * ROWS_PER_TILE
        with jax.named_scope("acc_zero"):
            pltpu.sync_copy(z_hbm.at[pl.ds(row0, ROWS_PER_TILE)],
                            acc_sh.at[pl.ds(row0, ROWS_PER_TILE)])
        base = lax.select(c == 0, s * CH_C0, 16 * CH_C0 + s * CH_C1)
        with jax.named_scope("zero_barrier"):
            plsc.subcore_barrier()

        sems = (sem0, sem1)

        def gather(j, b):
            return pltpu.async_copy(
                g_hbm.at[src_v.at[j]], rows_v.at[b], sems[b])

        def wait(j, b):
            pltpu.make_async_copy(
                g_hbm.at[src_v.at[j]], rows_v.at[b], sems[b]).wait()

        def scat(b, j):
            pltpu.sync_copy(rows_v.at[b], acc_sh.at[dst_v.at[j]],
                            add=True)

        def run(npasses):
            for p in range(npasses):
                sb = base + p * STAGE
                with jax.named_scope("stage_idx"):
                    pltpu.sync_copy(src_hbm.at[pl.ds(sb, STAGE)], src_v)
                    pltpu.sync_copy(dst_hbm.at[pl.ds(sb, STAGE)], dst_v)
                gather(0, 0)

                @pl.loop(0, STAGE // 2)
                def _(i):
                    j0 = 2 * i
                    wait(j0, 0)
                    gather(j0 + 1, 1)
                    scat(0, j0)
                    wait(j0 + 1, 1)

                    @pl.when(i < STAGE // 2 - 1)
                    def _():
                        gather(j0 + 2, 0)

                    scat(1, j0 + 1)

        with jax.named_scope("edge_loop"):
            @pl.when(c == 0)
            def _():
                run(CH_C0 // STAGE)

            @pl.when(c == 1)
            def _():
                run(CH_C1 // STAGE)

        with jax.named_scope("end_barrier"):
            plsc.subcore_barrier()
        with jax.named_scope("copyout"):
            pltpu.sync_copy(acc_sh.at[pl.ds(row0, ROWS_PER_TILE)],
                            out_hbm.at[c, pl.ds(row0, ROWS_PER_TILE)])

    return agg_kernel


def _tc_first_body(x_ref, w_ref, degp_ref, g_ref, dinv_ref):
    deg = 1.0 + degp_ref[0][:, 0:1] + degp_ref[1][:, 0:1]
    dinv = jax.lax.rsqrt(deg)
    p = jnp.dot(x_ref[...], w_ref[...],
                preferred_element_type=jnp.float32, precision=_PREC)
    g_ref[...] = dinv * p
    dinv_ref[...] = jnp.broadcast_to(dinv, (N8, 8))


def _tc_first(xp, w1p, degp):
    return pl.pallas_call(
        _tc_first_body,
        out_shape=(jax.ShapeDtypeStruct((N8, FF), jnp.float32),
                   jax.ShapeDtypeStruct((N8, 8), jnp.float32)),
    )(xp, w1p, degp)


def _tc_mid_body(pp_ref, g_ref, dinv_ref, b_ref, w_ref, out_ref):
    dinv = dinv_ref[...][:, 0:1]
    agg = pp_ref[0] + pp_ref[1] + g_ref[...]
    h = jnp.maximum(dinv * agg + b_ref[0:1, :], 0.0)
    out_ref[...] = dinv * jnp.dot(h, w_ref[...],
                                  preferred_element_type=jnp.float32,
                                  precision=_PREC)


def _tc_mid(pp, g, dinv, bp, wp):
    return pl.pallas_call(
        _tc_mid_body,
        out_shape=jax.ShapeDtypeStruct((N8, FF), jnp.float32),
    )(pp, g, dinv, bp, wp)


def _tc_final_body(pp_ref, g_ref, dinv_ref, b_ref, w_ref, bl_ref, out_ref):
    dinv = dinv_ref[...][:, 0:1]
    agg = pp_ref[0] + pp_ref[1] + g_ref[...]
    h = jnp.maximum(dinv * agg + b_ref[0:1, :], 0.0)
    out_ref[...] = jnp.dot(h, w_ref[...],
                           preferred_element_type=jnp.float32,
                           precision=_PREC) + bl_ref[0:1, :]


def _tc_final(pp, g, dinv, bp, wlp, blp):
    return pl.pallas_call(
        _tc_final_body,
        out_shape=jax.ShapeDtypeStruct((N8, 8), jnp.float32),
    )(pp, g, dinv, bp, wlp, blp)


def _padw(W, fi, fo):
    return jnp.zeros((fi, fo), jnp.float32).at[:W.shape[0], :W.shape[1]].set(W)


def _padb(b, fo):
    return jnp.broadcast_to(jnp.pad(b, (0, fo - b.shape[0])), (8, fo))


def kernel(x, edge_index, W1, b1, W2, b2, W6, b6, W3, b3, W4, b4, Wl, bl):
    f32 = jnp.float32
    src = edge_index[0]
    dst = edge_index[1]
    e = src.shape[0]
    pad = IDX_ROWS * CHUNK - e
    pad_dst = NN + (jnp.arange(pad, dtype=jnp.int32) % (N8 - NN))
    pad_src = jnp.arange(pad, dtype=jnp.int32) % NN
    src2d = jnp.concatenate(
        [src, pad_src]).reshape(IDX_ROWS, CHUNK)
    dst2d = jnp.concatenate(
        [dst, pad_dst]).reshape(IDX_ROWS, CHUNK)
    xp = jnp.zeros((N8, DD), f32).at[:NN].set(x)

    w1p = _padw(W1, DD, FF)
    w2p = _padw(W2, FF, FF)
    w6p = _padw(W6, FF, FF)
    w3p = _padw(W3, FF, FF)
    w4p = _padw(W4, FF, FF)
    wlp = _padw(Wl, FF, 8)
    b1p = _padb(b1, FF)
    b2p = _padb(b2, FF)
    b6p = _padb(b6, FF)
    b3p = _padb(b3, FF)
    b4p = _padb(b4, FF)
    blp = _padb(bl, 8)

    ones = jnp.ones((CHUNK, DEG_W), f32)
    zdeg = jnp.zeros((N8, DEG_W), f32)
    zf = jnp.zeros((N8, FF), f32)

    degp = _deg_kernel()(dst2d, ones, zdeg)
    g1, dinv = _tc_first(xp, w1p, degp)

    agg = _agg_kernel()
    p1 = agg(g1, src2d, dst2d, zf)
    g2 = _tc_mid(p1, g1, dinv, b1p, w2p)
    p2 = agg(g2, src2d, dst2d, zf)
    g3 = _tc_mid(p2, g2, dinv, b2p, w6p)
    p3 = agg(g3, src2d, dst2d, zf)
    g4 = _tc_mid(p3, g3, dinv, b6p, w3p)
    p4 = agg(g4, src2d, dst2d, zf)
    g5 = _tc_mid(p4, g4, dinv, b3p, w4p)
    p5 = agg(g5, src2d, dst2d, zf)
    out = _tc_final(p5, g5, dinv, b4p, wlp, blp)
    return out[:NN, :4]
